# Initial kernel scaffold; baseline (speedup 1.0000x reference)
import jax
import jax.numpy as jnp
from jax.experimental import pallas as pl


def kernel(x, tables):
    raise NotImplementedError("write your pallas kernel here")


# speedup vs baseline: 1.2081x; 1.2081x over previous
import functools

import jax
import jax.numpy as jnp
from jax import lax
from jax.experimental import pallas as pl
from jax.experimental.pallas import tpu as pltpu
from jax.experimental.pallas import tpu_sc as plsc

NUM_FIELDS = 26
VOCAB = 100000
EMBED_DIM = 32
BATCH = 16384

NC = 2
NS = 16
NW = NC * NS
LANES = 16

N = BATCH * NUM_FIELDS
PER_W = N // NW
CHUNK = 1664
NCHUNK = PER_W // CHUNK
KSTREAM = CHUNK // 128
NVEC = CHUNK // LANES


def _body(x_hbm, tab_hbm, out_hbm, offs_v, xv, idxv, rows_v, gsem):
    wid = lax.axis_index("s") * NC + lax.axis_index("c")
    base = wid * PER_W

    for i in range(NVEC):
        lanes = lax.iota(jnp.int32, LANES) + (i * LANES)
        offs_v[pl.ds(i * LANES, LANES)] = lax.rem(lanes, NUM_FIELDS) * VOCAB

    def chunk_body(g, carry):
        cbase = pl.multiple_of(base + g * CHUNK, CHUNK)
        pltpu.sync_copy(x_hbm.at[pl.ds(cbase, CHUNK)], xv)
        for i in range(NVEC):
            j, col = divmod(i * LANES, 128)
            s = pl.ds(i * LANES, LANES)
            idxv[j, pl.ds(col, LANES)] = xv[s] + offs_v[s]
        copies = []
        for j in range(KSTREAM):
            copies.append(
                pltpu.async_copy(
                    tab_hbm.at[idxv.at[j]],
                    rows_v.at[pl.ds(j * 128, 128), :],
                    gsem,
                )
            )
        for c in copies:
            c.wait()
        pltpu.sync_copy(rows_v, out_hbm.at[pl.ds(cbase, CHUNK), :])
        return carry

    lax.fori_loop(0, NCHUNK, chunk_body, 0)


_mesh = plsc.VectorSubcoreMesh(core_axis_name="c", subcore_axis_name="s")

_gather = functools.partial(
    pl.kernel,
    mesh=_mesh,
    out_type=jax.ShapeDtypeStruct((N, EMBED_DIM), jnp.float32),
    compiler_params=pltpu.CompilerParams(use_tc_tiling_on_sc=False),
    scratch_types=[
        pltpu.VMEM((CHUNK,), jnp.int32),
        pltpu.VMEM((CHUNK,), jnp.int32),
        pltpu.VMEM((KSTREAM, 128), jnp.int32),
        pltpu.VMEM((CHUNK, EMBED_DIM), jnp.float32),
        pltpu.SemaphoreType.DMA,
    ],
)(_body)


@jax.jit
def kernel(x, tables):
    x_flat = x.reshape(N).astype(jnp.int32)
    tab_flat = tables.reshape(NUM_FIELDS * VOCAB, EMBED_DIM)
    out = _gather(x_flat, tab_flat)
    return out.reshape(BATCH, NUM_FIELDS * EMBED_DIM)


# speedup vs baseline: 5.7476x; 4.7574x over previous
import functools

import jax
import jax.numpy as jnp
from jax import lax
from jax.experimental import pallas as pl
from jax.experimental.pallas import tpu as pltpu
from jax.experimental.pallas import tpu_sc as plsc

NUM_FIELDS = 26
VOCAB = 100000
EMBED_DIM = 32
BATCH = 16384

NC = 2
NS = 16
NW = NC * NS
LANES = 16

PAIRS = NUM_FIELDS * EMBED_DIM
PER_W = PAIRS // NW
HALF = BATCH // 2
UNROLL = 8
NGRP = HALF // (LANES * UNROLL)


def _body(xt_hbm, tab_hbm, out_hbm, row_v, idx_v, outb_v, sem):
    wid = lax.axis_index("s") * NC + lax.axis_index("c")

    def pair_body(k, carry):
        p = wid * PER_W + k
        i = p // EMBED_DIM
        e = p % EMBED_DIM

        @pl.when(jnp.logical_or(k == 0, i != (p - 1) // EMBED_DIM))
        def _():
            pltpu.sync_copy(xt_hbm.at[i, :], idx_v)

        pltpu.sync_copy(tab_hbm.at[i, e, :], row_v)

        for half in range(2):
            def grp(g, c2):
                for u in range(UNROLL):
                    off = g * (LANES * UNROLL) + u * LANES
                    iv = idx_v[pl.ds(half * HALF + off, LANES)]
                    outb_v[pl.ds(off, LANES)] = plsc.load_gather(row_v, [iv])
                return c2

            lax.fori_loop(0, NGRP, grp, 0)
            pltpu.sync_copy(outb_v, out_hbm.at[p, pl.ds(half * HALF, HALF)])
        return carry

    lax.fori_loop(0, PER_W, pair_body, 0)


_mesh = plsc.VectorSubcoreMesh(core_axis_name="c", subcore_axis_name="s")

_gather = functools.partial(
    pl.kernel,
    mesh=_mesh,
    out_type=jax.ShapeDtypeStruct((PAIRS, BATCH), jnp.float32),
    compiler_params=pltpu.CompilerParams(
        use_tc_tiling_on_sc=True, needs_layout_passes=False
    ),
    scratch_types=[
        pltpu.VMEM((VOCAB,), jnp.float32),
        pltpu.VMEM((BATCH,), jnp.int32),
        pltpu.VMEM((HALF,), jnp.float32),
        pltpu.SemaphoreType.DMA,
    ],
)(_body)


@jax.jit
def kernel(x, tables):
    xt = x.T
    tab_t = jnp.transpose(tables, (0, 2, 1))
    out_t = _gather(xt, tab_t)
    return out_t.T


# speedup vs baseline: 8.6044x; 1.4971x over previous
import functools

import jax
import jax.numpy as jnp
from jax import lax
from jax.experimental import pallas as pl
from jax.experimental.pallas import tpu as pltpu
from jax.experimental.pallas import tpu_sc as plsc

NUM_FIELDS = 26
VOCAB = 100000
EMBED_DIM = 32
BATCH = 16384

NC = 2
NS = 16
NW = NC * NS
LANES = 16

PAIRS = NUM_FIELDS * EMBED_DIM
PER_W = PAIRS // NW
QTR = BATCH // 4
NQ = 4


def _body(xt_hbm, tab_hbm, out_hbm, row_v, idx_v, ob0_v, ob1_v, sem0, sem1):
    wid = lax.axis_index("s") * NC + lax.axis_index("c")
    obufs = (ob0_v, ob1_v)
    sems = (sem0, sem1)

    def drain(b, p):
        pltpu.make_async_copy(
            obufs[b], out_hbm.at[p, pl.ds(0, QTR)], sems[b]
        ).wait()

    def pair_body(k, carry):
        p = wid * PER_W + k
        i = p // EMBED_DIM
        e = p % EMBED_DIM

        @pl.when(jnp.logical_or(k == 0, i != (p - 1) // EMBED_DIM))
        def _():
            pltpu.sync_copy(xt_hbm.at[i, :], idx_v)

        pltpu.sync_copy(tab_hbm.at[i, e, :], row_v)

        for q in range(NQ):
            b = q % 2
            if q < 2:
                @pl.when(k > 0)
                def _():
                    drain(b, p)
            else:
                drain(b, p)

            ob = obufs[b]
            qbase = q * QTR

            @plsc.parallel_loop(0, QTR, step=LANES, unroll=8)
            def _(g):
                iv = idx_v[pl.ds(qbase + g, LANES)]
                ob[pl.ds(g, LANES)] = plsc.load_gather(row_v, [iv])

            pltpu.async_copy(ob, out_hbm.at[p, pl.ds(qbase, QTR)], sems[b])
        return carry

    lax.fori_loop(0, PER_W, pair_body, 0)
    drain(0, wid * PER_W)
    drain(1, wid * PER_W)


_mesh = plsc.VectorSubcoreMesh(core_axis_name="c", subcore_axis_name="s")

_gather = functools.partial(
    pl.kernel,
    mesh=_mesh,
    out_type=jax.ShapeDtypeStruct((PAIRS, BATCH), jnp.float32),
    compiler_params=pltpu.CompilerParams(
        use_tc_tiling_on_sc=True, needs_layout_passes=False
    ),
    scratch_types=[
        pltpu.VMEM((VOCAB,), jnp.float32),
        pltpu.VMEM((BATCH,), jnp.int32),
        pltpu.VMEM((QTR,), jnp.float32),
        pltpu.VMEM((QTR,), jnp.float32),
        pltpu.SemaphoreType.DMA,
        pltpu.SemaphoreType.DMA,
    ],
)(_body)


@jax.jit
def kernel(x, tables):
    xt = x.T
    tab_t = jnp.transpose(tables, (0, 2, 1))
    out_t = _gather(xt, tab_t)
    return out_t.T
